# Initial kernel scaffold; baseline (speedup 1.0000x reference)
import jax
import jax.numpy as jnp
from jax.experimental import pallas as pl


def kernel(x, edge_index, W1_self, W1_neigh, b1, W2_self, W2_neigh, b2):
    raise NotImplementedError("write your pallas kernel here")


# speedup vs baseline: 2.3973x; 2.3973x over previous
import functools

import jax
import jax.numpy as jnp
from jax import lax
from jax.experimental import pallas as pl
from jax.experimental.pallas import tpu as pltpu
from jax.experimental.pallas import tpu_sc as plsc

N = 10000
E = 320000
D = 128
DA = 144

NC = 2
NS = 16
NW = NC * NS
EPW = E // NW
CH = 80
NCHUNK = EPW // CH
NP = 10240
RPS = NP // NS

F32 = jnp.float32


def _agg_body(xa, src, dst, zeros_a, agg_out, acc, sidx, didx, rows, sem):
    c = lax.axis_index("c")
    s = lax.axis_index("s")
    w = c * NS + s
    pltpu.sync_copy(zeros_a.at[pl.ds(s * RPS, RPS)], acc.at[pl.ds(s * RPS, RPS)])
    plsc.subcore_barrier()

    base = w * EPW

    def chunk(j, carry):
        off = base + j * CH
        pltpu.sync_copy(src.at[pl.ds(off, CH)], sidx)
        pltpu.sync_copy(dst.at[pl.ds(off, CH)], didx)
        pltpu.async_copy(xa.at[sidx], rows, sem).wait()
        pltpu.sync_copy(rows, acc.at[didx], add=True)
        return carry

    lax.fori_loop(0, NCHUNK, chunk, 0)
    plsc.subcore_barrier()
    pltpu.sync_copy(acc.at[pl.ds(s * RPS, RPS)], agg_out.at[c, pl.ds(s * RPS, RPS)])


def _sc_aggregate(xa, src, dst, zeros_a):
    mesh = plsc.VectorSubcoreMesh(core_axis_name="c", subcore_axis_name="s")
    kern = pl.kernel(
        _agg_body,
        out_type=jax.ShapeDtypeStruct((NC, NP, DA), F32),
        mesh=mesh,
        compiler_params=pltpu.CompilerParams(use_tc_tiling_on_sc=False, needs_layout_passes=False),
        scratch_types=[
            pltpu.VMEM_SHARED((NP, DA), F32),
            pltpu.VMEM((CH,), jnp.int32),
            pltpu.VMEM((CH,), jnp.int32),
            pltpu.VMEM((CH, DA), F32),
            pltpu.SemaphoreType.DMA,
        ],
    )
    return kern(xa, src, dst, zeros_a)


def _linear_body(aug_out, x_ref, aggp_ref, ws_ref, wn_ref, b_ref, o_ref):
    x = x_ref[...][:, :D]
    aggs = aggp_ref[0] + aggp_ref[1]
    agg = aggs[:, :D]
    deg = aggs[:, D:D + 1]
    rdeg = 1.0 / jnp.maximum(deg, 1.0)
    h = (jnp.dot(x, ws_ref[...], preferred_element_type=F32)
         + jnp.dot(agg * rdeg, wn_ref[...], preferred_element_type=F32)
         + b_ref[...][None, :])
    if aug_out:
        blk = h.shape[0]
        ones = jnp.ones((blk, 1), F32)
        pad = jnp.zeros((blk, DA - D - 1), F32)
        o_ref[...] = jnp.concatenate([h, ones, pad], axis=1)
    else:
        o_ref[...] = h


def _tc_linear(xa, aggp, ws_t, wn_t, b, aug_out):
    br = 2000
    grid = (N // br,)
    out_w = DA if aug_out else D
    return pl.pallas_call(
        functools.partial(_linear_body, aug_out),
        grid=grid,
        in_specs=[
            pl.BlockSpec((br, DA), lambda i: (i, 0)),
            pl.BlockSpec((NC, br, DA), lambda i: (0, i, 0)),
            pl.BlockSpec((D, D), lambda i: (0, 0)),
            pl.BlockSpec((D, D), lambda i: (0, 0)),
            pl.BlockSpec((D,), lambda i: (0,)),
        ],
        out_specs=pl.BlockSpec((br, out_w), lambda i: (i, 0)),
        out_shape=jax.ShapeDtypeStruct((N, out_w), F32),
    )(xa, aggp, ws_t, wn_t, b)


def _dot_body(h, src, dst, raw, mnw, mxw,
              sidx, didx, abuf, bbuf, obuf, statbuf, sem, sem2):
    c = lax.axis_index("c")
    s = lax.axis_index("s")
    w = c * NS + s
    base = w * EPW
    lanes = lax.iota(jnp.int32, 16)

    def chunk(j, carry):
        mn, mx = carry
        off = base + j * CH
        pltpu.sync_copy(src.at[pl.ds(off, CH)], sidx)
        pltpu.sync_copy(dst.at[pl.ds(off, CH)], didx)
        ca = pltpu.async_copy(h.at[sidx], abuf, sem)
        cb = pltpu.async_copy(h.at[didx], bbuf, sem2)
        ca.wait()
        cb.wait()

        def group(g, carry2):
            mn2, mx2 = carry2
            rowi = g * 16 + lanes
            acc = jnp.zeros((16,), F32)
            for d in range(D):
                col = jnp.full((16,), d, jnp.int32)
                va = plsc.load_gather(abuf, [rowi, col])
                vb = plsc.load_gather(bbuf, [rowi, col])
                acc = acc + va * vb
            obuf[pl.ds(g * 16, 16)] = acc
            return (jnp.minimum(mn2, acc), jnp.maximum(mx2, acc))

        mn, mx = lax.fori_loop(0, CH // 16, group, (mn, mx))
        pltpu.sync_copy(obuf, raw.at[pl.ds(off, CH)])
        return (mn, mx)

    mn, mx = lax.fori_loop(0, NCHUNK, chunk,
                           (jnp.full((16,), jnp.inf, F32),
                            jnp.full((16,), -jnp.inf, F32)))
    statbuf[...] = mn
    pltpu.sync_copy(statbuf, mnw.at[w])
    statbuf[...] = mx
    pltpu.sync_copy(statbuf, mxw.at[w])


def _sc_edge_dot(h, src, dst):
    mesh = plsc.VectorSubcoreMesh(core_axis_name="c", subcore_axis_name="s")
    kern = pl.kernel(
        _dot_body,
        compiler_params=pltpu.CompilerParams(use_tc_tiling_on_sc=False, needs_layout_passes=False),
        out_type=(
            jax.ShapeDtypeStruct((E,), F32),
            jax.ShapeDtypeStruct((NW, 16), F32),
            jax.ShapeDtypeStruct((NW, 16), F32),
        ),
        mesh=mesh,
        scratch_types=[
            pltpu.VMEM((CH,), jnp.int32),
            pltpu.VMEM((CH,), jnp.int32),
            pltpu.VMEM((CH, D), F32),
            pltpu.VMEM((CH, D), F32),
            pltpu.VMEM((CH,), F32),
            pltpu.VMEM((16,), F32),
            pltpu.SemaphoreType.DMA,
            pltpu.SemaphoreType.DMA,
        ],
    )
    return kern(h, src, dst)


def _norm_body(raw_ref, mnw_ref, mxw_ref, o_ref):
    mn = jnp.min(mnw_ref[...])
    mx = jnp.max(mxw_ref[...])
    o_ref[...] = (raw_ref[...] - mn) / (mx - mn)


def _tc_normalize(raw, mnw, mxw):
    rows = E // D
    raw2d = raw.reshape(rows, D)
    out2d = pl.pallas_call(
        _norm_body,
        out_shape=jax.ShapeDtypeStruct((rows, D), F32),
    )(raw2d, mnw, mxw)
    return out2d.reshape(E)


def kernel(x, edge_index, W1_self, W1_neigh, b1, W2_self, W2_neigh, b2):
    src = edge_index[0]
    dst = edge_index[1]
    ones_col = jnp.ones((N, 1), F32)
    pad = jnp.zeros((N, DA - D - 1), F32)
    xa = jnp.concatenate([x, ones_col, pad], axis=1)
    zeros_a = jnp.zeros((NP, DA), F32)

    aggp1 = _sc_aggregate(xa, src, dst, zeros_a)
    h1a = _tc_linear(xa, aggp1, W1_self.T, W1_neigh.T, b1, aug_out=True)
    aggp2 = _sc_aggregate(h1a, src, dst, zeros_a)
    h2 = _tc_linear(h1a, aggp2, W2_self.T, W2_neigh.T, b2, aug_out=False)

    raw, mnw, mxw = _sc_edge_dot(h2, src, dst)
    return _tc_normalize(raw, mnw, mxw)


# speedup vs baseline: 6.7728x; 2.8251x over previous
import functools

import jax
import jax.numpy as jnp
from jax import lax
from jax.experimental import pallas as pl
from jax.experimental.pallas import tpu as pltpu
from jax.experimental.pallas import tpu_sc as plsc

N = 10000
E = 320000
D = 128
DA = 144

NC = 2
NS = 16
NW = NC * NS
EPW = E // NW
CH = 80
NCHUNK = EPW // CH
NP = 10240
RPS = NP // NS

F32 = jnp.float32


def _agg_body(xa, src2d, dst2d, zeros_a, agg_out,
              acc, didx2d, si0, si1, rows0, rows1,
              gsem0, gsem1, isem0, isem1):
    c = lax.axis_index("c")
    s = lax.axis_index("s")
    w = c * NS + s
    cbase = w * NCHUNK
    pltpu.sync_copy(zeros_a.at[pl.ds(s * RPS, RPS)], acc.at[pl.ds(s * RPS, RPS)])
    pltpu.sync_copy(dst2d.at[pl.ds(cbase, NCHUNK)], didx2d)
    plsc.subcore_barrier()

    rbufs = (rows0, rows1)
    rsems = (gsem0, gsem1)
    ibufs = (si0, si1)
    isems = (isem0, isem1)

    pltpu.sync_copy(src2d.at[cbase], si0)
    pltpu.async_copy(xa.at[si0], rows0, gsem0)
    pltpu.async_copy(src2d.at[cbase + 1], si1, isem1)

    def pair(t, carry):
        for b in range(2):
            j = 2 * t + b
            pltpu.make_async_copy(xa.at[ibufs[b]], rbufs[b], rsems[b]).wait()
            pltpu.make_async_copy(src2d.at[cbase], ibufs[1 - b],
                                  isems[1 - b]).wait()
            pltpu.async_copy(xa.at[ibufs[1 - b]], rbufs[1 - b], rsems[1 - b])
            pltpu.async_copy(
                src2d.at[jnp.minimum(cbase + j + 2, cbase + NCHUNK - 1)],
                ibufs[b], isems[b])
            pltpu.sync_copy(rbufs[b], acc.at[didx2d.at[j]], add=True)
        return carry

    lax.fori_loop(0, (NCHUNK - 1) // 2, pair, 0)
    pltpu.make_async_copy(xa.at[ibufs[0]], rows0, gsem0).wait()
    pltpu.make_async_copy(src2d.at[cbase], si1, isem1).wait()
    pltpu.sync_copy(rows0, acc.at[didx2d.at[NCHUNK - 1]], add=True)

    plsc.subcore_barrier()
    pltpu.sync_copy(acc.at[pl.ds(s * RPS, RPS)], agg_out.at[c, pl.ds(s * RPS, RPS)])


def _sc_aggregate(xa, src2d, dst2d, zeros_a):
    mesh = plsc.VectorSubcoreMesh(core_axis_name="c", subcore_axis_name="s")
    kern = pl.kernel(
        _agg_body,
        out_type=jax.ShapeDtypeStruct((NC, NP, DA), F32),
        mesh=mesh,
        compiler_params=pltpu.CompilerParams(use_tc_tiling_on_sc=False, needs_layout_passes=False),
        scratch_types=[
            pltpu.VMEM_SHARED((NP, DA), F32),
            pltpu.VMEM((NCHUNK, CH), jnp.int32),
            pltpu.VMEM((CH,), jnp.int32),
            pltpu.VMEM((CH,), jnp.int32),
            pltpu.VMEM((CH, DA), F32),
            pltpu.VMEM((CH, DA), F32),
            pltpu.SemaphoreType.DMA,
            pltpu.SemaphoreType.DMA,
            pltpu.SemaphoreType.DMA,
            pltpu.SemaphoreType.DMA,
        ],
    )
    return kern(xa, src2d, dst2d, zeros_a)


def _linear_body(aug_out, x_ref, aggp_ref, ws_ref, wn_ref, b_ref, o_ref):
    x = x_ref[...][:, :D]
    aggs = aggp_ref[0] + aggp_ref[1]
    agg = aggs[:, :D]
    deg = aggs[:, D:D + 1]
    rdeg = 1.0 / jnp.maximum(deg, 1.0)
    h = (jnp.dot(x, ws_ref[...], preferred_element_type=F32)
         + jnp.dot(agg * rdeg, wn_ref[...], preferred_element_type=F32)
         + b_ref[...][None, :])
    if aug_out:
        blk = h.shape[0]
        ones = jnp.ones((blk, 1), F32)
        pad = jnp.zeros((blk, DA - D - 1), F32)
        o_ref[...] = jnp.concatenate([h, ones, pad], axis=1)
    else:
        o_ref[...] = h


def _tc_linear(xa, aggp, ws_t, wn_t, b, aug_out):
    br = 2000
    grid = (N // br,)
    out_w = DA if aug_out else D
    return pl.pallas_call(
        functools.partial(_linear_body, aug_out),
        grid=grid,
        in_specs=[
            pl.BlockSpec((br, DA), lambda i: (i, 0)),
            pl.BlockSpec((NC, br, DA), lambda i: (0, i, 0)),
            pl.BlockSpec((D, D), lambda i: (0, 0)),
            pl.BlockSpec((D, D), lambda i: (0, 0)),
            pl.BlockSpec((D,), lambda i: (0,)),
        ],
        out_specs=pl.BlockSpec((br, out_w), lambda i: (i, 0)),
        out_shape=jax.ShapeDtypeStruct((N, out_w), F32),
    )(xa, aggp, ws_t, wn_t, b)


def _dot_body(h, src2d, dst2d, raw, mnw, mxw,
              sidx2d, didx2d, a0, a1, b0, b1, obuf, statbuf,
              sa0, sa1, sb0, sb1):
    c = lax.axis_index("c")
    s = lax.axis_index("s")
    w = c * NS + s
    base = w * EPW
    lanes = lax.iota(jnp.int32, 16)

    pltpu.sync_copy(src2d.at[pl.ds(w * NCHUNK, NCHUNK)], sidx2d)
    pltpu.sync_copy(dst2d.at[pl.ds(w * NCHUNK, NCHUNK)], didx2d)

    abufs = (a0, a1)
    bbufs = (b0, b1)
    asems = (sa0, sa1)
    bsems = (sb0, sb1)

    def issue(j, b):
        pltpu.async_copy(h.at[sidx2d.at[j]], abufs[b], asems[b])
        pltpu.async_copy(h.at[didx2d.at[j]], bbufs[b], bsems[b])

    def wait(j, b):
        pltpu.make_async_copy(h.at[sidx2d.at[j]], abufs[b], asems[b]).wait()
        pltpu.make_async_copy(h.at[didx2d.at[j]], bbufs[b], bsems[b]).wait()

    def compute(j, b, mn, mx):
        def group(g, carry2):
            mn2, mx2 = carry2
            rowi = g * 16 + lanes
            acc = jnp.zeros((16,), F32)
            for t in range(D):
                col = lanes + t
                col = jnp.where(col >= D, col - D, col)
                va = plsc.load_gather(abufs[b], [rowi, col])
                vb = plsc.load_gather(bbufs[b], [rowi, col])
                acc = acc + va * vb
            obuf[pl.ds(g * 16, 16)] = acc
            return (jnp.minimum(mn2, acc), jnp.maximum(mx2, acc))

        mn, mx = lax.fori_loop(0, CH // 16, group, (mn, mx))
        pltpu.sync_copy(obuf, raw.at[pl.ds(base + j * CH, CH)])
        return mn, mx

    issue(0, 0)

    def pair(t, carry):
        mn, mx = carry
        for b in range(2):
            j = 2 * t + b
            wait(j, b)
            issue(j + 1, 1 - b)
            mn, mx = compute(j, b, mn, mx)
        return (mn, mx)

    mn, mx = lax.fori_loop(0, (NCHUNK - 1) // 2, pair,
                           (jnp.full((16,), jnp.inf, F32),
                            jnp.full((16,), -jnp.inf, F32)))
    wait(NCHUNK - 1, 0)
    mn, mx = compute(NCHUNK - 1, 0, mn, mx)

    statbuf[...] = mn
    pltpu.sync_copy(statbuf, mnw.at[w])
    statbuf[...] = mx
    pltpu.sync_copy(statbuf, mxw.at[w])


def _sc_edge_dot(h, src2d, dst2d):
    mesh = plsc.VectorSubcoreMesh(core_axis_name="c", subcore_axis_name="s")
    kern = pl.kernel(
        _dot_body,
        compiler_params=pltpu.CompilerParams(use_tc_tiling_on_sc=False, needs_layout_passes=False),
        out_type=(
            jax.ShapeDtypeStruct((E,), F32),
            jax.ShapeDtypeStruct((NW, 16), F32),
            jax.ShapeDtypeStruct((NW, 16), F32),
        ),
        mesh=mesh,
        scratch_types=[
            pltpu.VMEM((NCHUNK, CH), jnp.int32),
            pltpu.VMEM((NCHUNK, CH), jnp.int32),
            pltpu.VMEM((CH, D), F32),
            pltpu.VMEM((CH, D), F32),
            pltpu.VMEM((CH, D), F32),
            pltpu.VMEM((CH, D), F32),
            pltpu.VMEM((CH,), F32),
            pltpu.VMEM((16,), F32),
            pltpu.SemaphoreType.DMA,
            pltpu.SemaphoreType.DMA,
            pltpu.SemaphoreType.DMA,
            pltpu.SemaphoreType.DMA,
        ],
    )
    return kern(h, src2d, dst2d)


def _norm_body(raw_ref, mnw_ref, mxw_ref, o_ref):
    mn = jnp.min(mnw_ref[...])
    mx = jnp.max(mxw_ref[...])
    o_ref[...] = (raw_ref[...] - mn) / (mx - mn)


def _tc_normalize(raw, mnw, mxw):
    rows = E // D
    raw2d = raw.reshape(rows, D)
    out2d = pl.pallas_call(
        _norm_body,
        out_shape=jax.ShapeDtypeStruct((rows, D), F32),
    )(raw2d, mnw, mxw)
    return out2d.reshape(E)


def kernel(x, edge_index, W1_self, W1_neigh, b1, W2_self, W2_neigh, b2):
    src2d = edge_index[0].reshape(E // CH, CH)
    dst2d = edge_index[1].reshape(E // CH, CH)
    ones_col = jnp.ones((N, 1), F32)
    pad = jnp.zeros((N, DA - D - 1), F32)
    xa = jnp.concatenate([x, ones_col, pad], axis=1)
    zeros_a = jnp.zeros((NP, DA), F32)

    aggp1 = _sc_aggregate(xa, src2d, dst2d, zeros_a)
    h1a = _tc_linear(xa, aggp1, W1_self.T, W1_neigh.T, b1, aug_out=True)
    aggp2 = _sc_aggregate(h1a, src2d, dst2d, zeros_a)
    h2 = _tc_linear(h1a, aggp2, W2_self.T, W2_neigh.T, b2, aug_out=False)

    raw, mnw, mxw = _sc_edge_dot(h2, src2d, dst2d)
    return _tc_normalize(raw, mnw, mxw)


# speedup vs baseline: 8.5038x; 1.2556x over previous
import functools

import jax
import jax.numpy as jnp
from jax import lax
from jax.experimental import pallas as pl
from jax.experimental.pallas import tpu as pltpu
from jax.experimental.pallas import tpu_sc as plsc

N = 10000
E = 320000
D = 128
DA = 144

NC = 2
NS = 16
NW = NC * NS
EPW = E // NW
CH = 80
NCHUNK = EPW // CH
NP = 10240
RPS = NP // NS

F32 = jnp.float32


def _agg_body(xa, src2d, dst2d, zeros_a, agg_out,
              acc, didx2d, si0, si1, rows0, rows1,
              gsem0, gsem1, isem0, isem1):
    c = lax.axis_index("c")
    s = lax.axis_index("s")
    w = c * NS + s
    cbase = w * NCHUNK
    pltpu.sync_copy(zeros_a.at[pl.ds(s * RPS, RPS)], acc.at[pl.ds(s * RPS, RPS)])
    pltpu.sync_copy(dst2d.at[pl.ds(cbase, NCHUNK)], didx2d)
    plsc.subcore_barrier()

    rbufs = (rows0, rows1)
    rsems = (gsem0, gsem1)
    ibufs = (si0, si1)
    isems = (isem0, isem1)

    pltpu.sync_copy(src2d.at[cbase], si0)
    pltpu.async_copy(xa.at[si0], rows0, gsem0)
    pltpu.async_copy(src2d.at[cbase + 1], si1, isem1)

    def pair(t, carry):
        for b in range(2):
            j = 2 * t + b
            pltpu.make_async_copy(xa.at[ibufs[b]], rbufs[b], rsems[b]).wait()
            pltpu.make_async_copy(src2d.at[cbase], ibufs[1 - b],
                                  isems[1 - b]).wait()
            pltpu.async_copy(xa.at[ibufs[1 - b]], rbufs[1 - b], rsems[1 - b])
            pltpu.async_copy(
                src2d.at[jnp.minimum(cbase + j + 2, cbase + NCHUNK - 1)],
                ibufs[b], isems[b])
            pltpu.sync_copy(rbufs[b], acc.at[didx2d.at[j]], add=True)
        return carry

    lax.fori_loop(0, (NCHUNK - 1) // 2, pair, 0)
    pltpu.make_async_copy(xa.at[ibufs[0]], rows0, gsem0).wait()
    pltpu.make_async_copy(src2d.at[cbase], si1, isem1).wait()
    pltpu.sync_copy(rows0, acc.at[didx2d.at[NCHUNK - 1]], add=True)

    plsc.subcore_barrier()
    pltpu.sync_copy(acc.at[pl.ds(s * RPS, RPS)], agg_out.at[c, pl.ds(s * RPS, RPS)])


def _sc_aggregate(xa, src2d, dst2d, zeros_a):
    mesh = plsc.VectorSubcoreMesh(core_axis_name="c", subcore_axis_name="s")
    kern = pl.kernel(
        _agg_body,
        out_type=jax.ShapeDtypeStruct((NC, NP, DA), F32),
        mesh=mesh,
        compiler_params=pltpu.CompilerParams(use_tc_tiling_on_sc=False, needs_layout_passes=False),
        scratch_types=[
            pltpu.VMEM_SHARED((NP, DA), F32),
            pltpu.VMEM((NCHUNK, CH), jnp.int32),
            pltpu.VMEM((CH,), jnp.int32),
            pltpu.VMEM((CH,), jnp.int32),
            pltpu.VMEM((CH, DA), F32),
            pltpu.VMEM((CH, DA), F32),
            pltpu.SemaphoreType.DMA,
            pltpu.SemaphoreType.DMA,
            pltpu.SemaphoreType.DMA,
            pltpu.SemaphoreType.DMA,
        ],
    )
    return kern(xa, src2d, dst2d, zeros_a)


def _linear_body(aug_out, x_ref, aggp_ref, ws_ref, wn_ref, b_ref, o_ref):
    x = x_ref[...][:, :D]
    aggs = aggp_ref[0] + aggp_ref[1]
    agg = aggs[:, :D]
    deg = aggs[:, D:D + 1]
    rdeg = 1.0 / jnp.maximum(deg, 1.0)
    h = (jnp.dot(x, ws_ref[...], preferred_element_type=F32)
         + jnp.dot(agg * rdeg, wn_ref[...], preferred_element_type=F32)
         + b_ref[...][None, :])
    if aug_out:
        blk = h.shape[0]
        ones = jnp.ones((blk, 1), F32)
        pad = jnp.zeros((blk, DA - D - 1), F32)
        o_ref[...] = jnp.concatenate([h, ones, pad], axis=1)
    else:
        o_ref[...] = h


def _tc_linear(xa, aggp, ws_t, wn_t, b, aug_out):
    br = 2000
    grid = (N // br,)
    out_w = DA if aug_out else D
    return pl.pallas_call(
        functools.partial(_linear_body, aug_out),
        grid=grid,
        in_specs=[
            pl.BlockSpec((br, DA), lambda i: (i, 0)),
            pl.BlockSpec((NC, br, DA), lambda i: (0, i, 0)),
            pl.BlockSpec((D, D), lambda i: (0, 0)),
            pl.BlockSpec((D, D), lambda i: (0, 0)),
            pl.BlockSpec((D,), lambda i: (0,)),
        ],
        out_specs=pl.BlockSpec((br, out_w), lambda i: (i, 0)),
        out_shape=jax.ShapeDtypeStruct((N, out_w), F32),
    )(xa, aggp, ws_t, wn_t, b)


def _dot_body(hi, src2d, dst2d, raw, mnw, mxw,
              sidx2d, didx2d, a0, a1, b0, b1, obuf, statbuf,
              sa0, sa1, sb0, sb1):
    c = lax.axis_index("c")
    s = lax.axis_index("s")
    w = c * NS + s
    base = w * EPW
    lanes = lax.iota(jnp.int32, 16)
    DW = D // 2

    pltpu.sync_copy(src2d.at[pl.ds(w * NCHUNK, NCHUNK)], sidx2d)
    pltpu.sync_copy(dst2d.at[pl.ds(w * NCHUNK, NCHUNK)], didx2d)

    abufs = (a0, a1)
    bbufs = (b0, b1)
    asems = (sa0, sa1)
    bsems = (sb0, sb1)

    def issue(j, b):
        pltpu.async_copy(hi.at[sidx2d.at[j]], abufs[b], asems[b])
        pltpu.async_copy(hi.at[didx2d.at[j]], bbufs[b], bsems[b])

    def wait(j, b):
        pltpu.make_async_copy(hi.at[sidx2d.at[j]], abufs[b], asems[b]).wait()
        pltpu.make_async_copy(hi.at[didx2d.at[j]], bbufs[b], bsems[b]).wait()

    def compute(j, b, mn, mx):
        def group(g, carry2):
            mn2, mx2 = carry2
            rowi = g * 16 + lanes
            acc = jnp.zeros((16,), F32)
            for t in range(DW):
                col = lanes + t
                col = jnp.where(col >= DW, col - DW, col)
                va = plsc.load_gather(abufs[b], [rowi, col])
                vb = plsc.load_gather(bbufs[b], [rowi, col])
                prod = plsc.bitcast(va, jnp.bfloat16) * plsc.bitcast(vb, jnp.bfloat16)
                p0, p1 = plsc.unpack(prod, format=plsc.PackFormat.INTERLEAVED)
                acc = acc + (p0.astype(F32) + p1.astype(F32))
            obuf[pl.ds(j * CH + g * 16, 16)] = acc
            return (jnp.minimum(mn2, acc), jnp.maximum(mx2, acc))

        return lax.fori_loop(0, CH // 16, group, (mn, mx))

    issue(0, 0)

    def pair(t, carry):
        mn, mx = carry
        for b in range(2):
            j = 2 * t + b
            wait(j, b)
            issue(j + 1, 1 - b)
            mn, mx = compute(j, b, mn, mx)
        return (mn, mx)

    mn, mx = lax.fori_loop(0, (NCHUNK - 1) // 2, pair,
                           (jnp.full((16,), jnp.inf, F32),
                            jnp.full((16,), -jnp.inf, F32)))
    wait(NCHUNK - 1, 0)
    mn, mx = compute(NCHUNK - 1, 0, mn, mx)

    pltpu.sync_copy(obuf, raw.at[pl.ds(base, EPW)])
    statbuf[...] = mn
    pltpu.sync_copy(statbuf, mnw.at[w])
    statbuf[...] = mx
    pltpu.sync_copy(statbuf, mxw.at[w])


def _sc_edge_dot(hi, src2d, dst2d):
    mesh = plsc.VectorSubcoreMesh(core_axis_name="c", subcore_axis_name="s")
    kern = pl.kernel(
        _dot_body,
        compiler_params=pltpu.CompilerParams(use_tc_tiling_on_sc=False, needs_layout_passes=False),
        out_type=(
            jax.ShapeDtypeStruct((E,), F32),
            jax.ShapeDtypeStruct((NW, 16), F32),
            jax.ShapeDtypeStruct((NW, 16), F32),
        ),
        mesh=mesh,
        scratch_types=[
            pltpu.VMEM((NCHUNK, CH), jnp.int32),
            pltpu.VMEM((NCHUNK, CH), jnp.int32),
            pltpu.VMEM((CH, D // 2), jnp.int32),
            pltpu.VMEM((CH, D // 2), jnp.int32),
            pltpu.VMEM((CH, D // 2), jnp.int32),
            pltpu.VMEM((CH, D // 2), jnp.int32),
            pltpu.VMEM((EPW,), F32),
            pltpu.VMEM((16,), F32),
            pltpu.SemaphoreType.DMA,
            pltpu.SemaphoreType.DMA,
            pltpu.SemaphoreType.DMA,
            pltpu.SemaphoreType.DMA,
        ],
    )
    return kern(hi, src2d, dst2d)


def _norm_body(raw_ref, mnw_ref, mxw_ref, o_ref):
    mn = jnp.min(mnw_ref[...])
    mx = jnp.max(mxw_ref[...])
    o_ref[...] = (raw_ref[...] - mn) / (mx - mn)


def _tc_normalize(raw, mnw, mxw):
    rows = E // D
    raw2d = raw.reshape(rows, D)
    out2d = pl.pallas_call(
        _norm_body,
        out_shape=jax.ShapeDtypeStruct((rows, D), F32),
    )(raw2d, mnw, mxw)
    return out2d.reshape(E)


def kernel(x, edge_index, W1_self, W1_neigh, b1, W2_self, W2_neigh, b2):
    src2d = edge_index[0].reshape(E // CH, CH)
    dst2d = edge_index[1].reshape(E // CH, CH)
    ones_col = jnp.ones((N, 1), F32)
    pad = jnp.zeros((N, DA - D - 1), F32)
    xa = jnp.concatenate([x, ones_col, pad], axis=1)
    zeros_a = jnp.zeros((NP, DA), F32)

    aggp1 = _sc_aggregate(xa, src2d, dst2d, zeros_a)
    h1a = _tc_linear(xa, aggp1, W1_self.T, W1_neigh.T, b1, aug_out=True)
    aggp2 = _sc_aggregate(h1a, src2d, dst2d, zeros_a)
    h2 = _tc_linear(h1a, aggp2, W2_self.T, W2_neigh.T, b2, aug_out=False)

    h2i = lax.bitcast_convert_type(
        h2.astype(jnp.bfloat16).reshape(N, D // 2, 2), jnp.int32)
    raw, mnw, mxw = _sc_edge_dot(h2i, src2d, dst2d)
    return _tc_normalize(raw, mnw, mxw)
